# Initial kernel scaffold; baseline (speedup 1.0000x reference)
import functools

import jax
import jax.numpy as jnp
from jax import lax
from jax.experimental import pallas as pl
from jax.experimental.pallas import tpu as pltpu
from jax.experimental.pallas import tpu_sc as plsc

N = 10000
E = 160000
D = 128
ROW = 144
NT = 16
E_PAD = 163840
EPT = E_PAD // NT
CH = 128
NCH = EPT // CH
RPT = N // NT
WCOL = 128


def _sc_aggregate(x2, src, dst, cnt, zeros):
    mesh = plsc.VectorSubcoreMesh(core_axis_name="c", subcore_axis_name="s")

    @functools.partial(
        pl.kernel,
        out_type=jax.ShapeDtypeStruct((2 * N, ROW), jnp.float32),
        mesh=mesh,
        scratch_types=[
            pltpu.VMEM((CH,), jnp.int32),
            pltpu.VMEM((CH,), jnp.int32),
            pltpu.VMEM((CH,), jnp.int32),
            pltpu.VMEM((CH,), jnp.float32),
            pltpu.VMEM((CH, D), jnp.float32),
            pltpu.VMEM((CH, ROW), jnp.float32),
            pltpu.VMEM_SHARED((N, ROW), jnp.float32),
            pltpu.SemaphoreType.DMA,
        ],
    )
    def agg(x2_hbm, src_hbm, dst_hbm, cnt_hbm, z_hbm, out_hbm,
            src_v, idx_v, dst_v, cnt_v, rows_g, rows_s, acc, sem):
        c = lax.axis_index("c")
        s = lax.axis_index("s")
        pltpu.sync_copy(z_hbm, acc.at[pl.ds(s * RPT, RPT)])
        pltpu.sync_copy(z_hbm.at[pl.ds(0, CH)], rows_s)
        plsc.subcore_barrier()

        base = s * EPT

        def chunk(i, carry):
            off = base + i * CH
            pltpu.sync_copy(src_hbm.at[pl.ds(off, CH)], src_v)
            pltpu.sync_copy(dst_hbm.at[pl.ds(off, CH)], dst_v)
            pltpu.sync_copy(cnt_hbm.at[pl.ds(off, CH)], cnt_v)
            for g in range(CH // 16):
                sl = pl.ds(g * 16, 16)
                idx_v[sl] = src_v[sl] * 2 + c
            pltpu.async_copy(x2_hbm.at[idx_v], rows_g, sem).wait()

            def edge(e, carry2):
                cv = cnt_v[e]
                rows_s[e, WCOL] = cv
                for f in range(D // 16):
                    fsl = pl.ds(f * 16, 16)
                    rows_s[e, fsl] = rows_g[e, fsl] * cv
                return carry2

            lax.fori_loop(0, CH, edge, 0)
            pltpu.sync_copy(rows_s, acc.at[dst_v], add=True)
            return carry

        lax.fori_loop(0, NCH, chunk, 0)
        plsc.subcore_barrier()
        pltpu.sync_copy(acc.at[pl.ds(s * RPT, RPT)],
                        out_hbm.at[pl.ds(c * N + s * RPT, RPT)])

    return agg(x2, src, dst, cnt, zeros)


def _tc_layer(a0, a1, h, wn0, wn1, wh, b):
    br = 1000

    def body(a0_r, a1_r, h_r, wn0_r, wn1_r, wh_r, b_r, o_r):
        a0b = a0_r[...]
        a1b = a1_r[...]
        w = a0b[:, WCOL:WCOL + 1] + a1b[:, WCOL:WCOL + 1]
        inv = 1.0 / jnp.maximum(w, 1.0)
        n0 = a0b[:, :D] * inv
        n1 = a1b[:, :D] * inv
        z = (jnp.dot(n0, wn0_r[...], preferred_element_type=jnp.float32)
             + jnp.dot(n1, wn1_r[...], preferred_element_type=jnp.float32)
             + jnp.dot(h_r[...], wh_r[...], preferred_element_type=jnp.float32)
             + b_r[...])
        z = jnp.maximum(z, 0.0)
        ssum = jnp.sum(z * z, axis=1, keepdims=True)
        o_r[...] = z * lax.rsqrt(jnp.where(ssum == 0.0, 1.0, ssum))

    return pl.pallas_call(
        body,
        grid=(N // br,),
        in_specs=[
            pl.BlockSpec((br, ROW), lambda i: (i, 0)),
            pl.BlockSpec((br, ROW), lambda i: (i, 0)),
            pl.BlockSpec((br, 2 * D), lambda i: (i, 0)),
            pl.BlockSpec((D, 2 * D), lambda i: (0, 0)),
            pl.BlockSpec((D, 2 * D), lambda i: (0, 0)),
            pl.BlockSpec((2 * D, 2 * D), lambda i: (0, 0)),
            pl.BlockSpec((1, 2 * D), lambda i: (0, 0)),
        ],
        out_specs=pl.BlockSpec((br, 2 * D), lambda i: (i, 0)),
        out_shape=jax.ShapeDtypeStruct((N, 2 * D), jnp.float32),
    )(a0, a1, h, wn0, wn1, wh, b.reshape(1, 2 * D))


def kernel(x, edge_index, edge_count, W1, b1, W2, b2):
    src = edge_index[0].astype(jnp.int32)
    dst = edge_index[1].astype(jnp.int32)
    cnt = edge_count.astype(jnp.float32)
    pad = E_PAD - E
    src_p = jnp.concatenate([src, jnp.zeros((pad,), jnp.int32)])
    dst_p = jnp.concatenate([dst, jnp.zeros((pad,), jnp.int32)])
    cnt_p = jnp.concatenate([cnt, jnp.zeros((pad,), jnp.float32)])
    zeros = jnp.zeros((RPT, ROW), jnp.float32)

    def layer(h, W, b):
        agg = _sc_aggregate(h.reshape(2 * N, D), src_p, dst_p, cnt_p, zeros)
        return _tc_layer(agg[:N], agg[N:], h, W[:D], W[D:2 * D], W[2 * D:], b)

    h1 = layer(x, W1, b1)
    return layer(h1, W2, b2)


# speedup vs baseline: 1.9861x; 1.9861x over previous
import functools

import jax
import jax.numpy as jnp
from jax import lax
from jax.experimental import pallas as pl
from jax.experimental.pallas import tpu as pltpu
from jax.experimental.pallas import tpu_sc as plsc

N = 10000
E = 160000
D = 128
NC = 2
NT = 16
E_PAD = 163840
EPT = E_PAD // NT
CH = 128
NCH = EPT // CH
WPT = E_PAD // (NC * NT)
WCH = WPT // CH
N_PAD = 10240
RPT = N_PAD // NT


def _sc_aggregate(x2, src, dst, cnt, zeros, with_w):
    mesh = plsc.VectorSubcoreMesh(core_axis_name="c", subcore_axis_name="s")
    out_rows = (4 if with_w else 2) * N_PAD

    @functools.partial(
        pl.kernel,
        out_type=jax.ShapeDtypeStruct((out_rows, D), jnp.float32),
        mesh=mesh,
        scratch_types=[
            pltpu.VMEM((CH,), jnp.int32),
            pltpu.VMEM((CH,), jnp.int32),
            pltpu.VMEM((CH,), jnp.int32),
            pltpu.VMEM((CH + 16,), jnp.float32),
            pltpu.VMEM((CH, D), jnp.float32),
            pltpu.VMEM_SHARED((N_PAD, D), jnp.float32),
            pltpu.SemaphoreType.DMA,
        ],
    )
    def agg(x2_hbm, src_hbm, dst_hbm, cnt_hbm, z_hbm, out_hbm,
            src_v, idx_v, dst_v, cnt_v, rows_v, acc, sem):
        c = lax.axis_index("c")
        s = lax.axis_index("s")
        pltpu.sync_copy(z_hbm, acc.at[pl.ds(s * RPT, RPT)])
        plsc.subcore_barrier()

        base = s * EPT
        cvec = jnp.full((16,), c, dtype=jnp.int32)

        def chunk(i, carry):
            off = base + i * CH
            pltpu.sync_copy(src_hbm.at[pl.ds(off, CH)], src_v)
            pltpu.sync_copy(dst_hbm.at[pl.ds(off, CH)], dst_v)
            pltpu.sync_copy(cnt_hbm.at[pl.ds(off, CH)], cnt_v.at[pl.ds(0, CH)])
            for g in range(CH // 16):
                sl = pl.ds(g * 16, 16)
                idx_v[sl] = src_v[sl] * 2 + cvec
            pltpu.async_copy(x2_hbm.at[idx_v], rows_v, sem).wait()

            def edge(e, carry2):
                cv = cnt_v[pl.ds(e, 16)][0]
                cvv = jnp.full((16,), cv, dtype=jnp.float32)
                for f in range(D // 16):
                    fsl = pl.ds(f * 16, 16)
                    rows_v[e, fsl] = rows_v[e, fsl] * cvv
                return carry2

            lax.fori_loop(0, CH, edge, 0)
            pltpu.sync_copy(rows_v, acc.at[dst_v], add=True)
            return carry

        lax.fori_loop(0, NCH, chunk, 0)
        plsc.subcore_barrier()
        pltpu.sync_copy(acc.at[pl.ds(s * RPT, RPT)],
                        out_hbm.at[pl.ds(c * N_PAD + s * RPT, RPT)])

        if with_w:
            plsc.subcore_barrier()
            pltpu.sync_copy(z_hbm, acc.at[pl.ds(s * RPT, RPT)])
            pltpu.sync_copy(z_hbm.at[pl.ds(0, CH)], rows_v)
            plsc.subcore_barrier()
            wbase = (s * NC + c) * WPT
            lane0 = jnp.where(lax.iota(jnp.int32, 16) == 0,
                              jnp.full((16,), 1.0, dtype=jnp.float32),
                              jnp.full((16,), 0.0, dtype=jnp.float32))

            def wchunk(i, carry):
                off = wbase + i * CH
                pltpu.sync_copy(dst_hbm.at[pl.ds(off, CH)], dst_v)
                pltpu.sync_copy(cnt_hbm.at[pl.ds(off, CH)],
                                cnt_v.at[pl.ds(0, CH)])

                def wedge(e, carry2):
                    cv = cnt_v[pl.ds(e, 16)][0]
                    rows_v[e, pl.ds(0, 16)] = lane0 * jnp.full(
                        (16,), cv, dtype=jnp.float32)
                    return carry2

                lax.fori_loop(0, CH, wedge, 0)
                pltpu.sync_copy(rows_v, acc.at[dst_v], add=True)
                return carry

            lax.fori_loop(0, WCH, wchunk, 0)
            plsc.subcore_barrier()
            pltpu.sync_copy(
                acc.at[pl.ds(s * RPT, RPT)],
                out_hbm.at[pl.ds((2 + c) * N_PAD + s * RPT, RPT)])

    return agg(x2, src, dst, cnt, zeros)


def _tc_layer(a0, a1, w0, w1, h, wn0, wn1, wh, b):
    br = 1000

    def body(a0_r, a1_r, w0_r, w1_r, h_r, wn0_r, wn1_r, wh_r, b_r, o_r):
        w = w0_r[:, :1] + w1_r[:, :1]
        inv = 1.0 / jnp.maximum(w, 1.0)
        n0 = a0_r[...] * inv
        n1 = a1_r[...] * inv
        z = (jnp.dot(n0, wn0_r[...], preferred_element_type=jnp.float32)
             + jnp.dot(n1, wn1_r[...], preferred_element_type=jnp.float32)
             + jnp.dot(h_r[...], wh_r[...], preferred_element_type=jnp.float32)
             + b_r[...])
        z = jnp.maximum(z, 0.0)
        ssum = jnp.sum(z * z, axis=1, keepdims=True)
        o_r[...] = z * lax.rsqrt(jnp.where(ssum == 0.0, 1.0, ssum))

    return pl.pallas_call(
        body,
        grid=(N // br,),
        in_specs=[
            pl.BlockSpec((br, D), lambda i: (i, 0)),
            pl.BlockSpec((br, D), lambda i: (i, 0)),
            pl.BlockSpec((br, D), lambda i: (i, 0)),
            pl.BlockSpec((br, D), lambda i: (i, 0)),
            pl.BlockSpec((br, 2 * D), lambda i: (i, 0)),
            pl.BlockSpec((D, 2 * D), lambda i: (0, 0)),
            pl.BlockSpec((D, 2 * D), lambda i: (0, 0)),
            pl.BlockSpec((2 * D, 2 * D), lambda i: (0, 0)),
            pl.BlockSpec((1, 2 * D), lambda i: (0, 0)),
        ],
        out_specs=pl.BlockSpec((br, 2 * D), lambda i: (i, 0)),
        out_shape=jax.ShapeDtypeStruct((N, 2 * D), jnp.float32),
    )(a0, a1, w0, w1, h, wn0, wn1, wh, b.reshape(1, 2 * D))


def kernel(x, edge_index, edge_count, W1, b1, W2, b2):
    src = edge_index[0].astype(jnp.int32)
    dst = edge_index[1].astype(jnp.int32)
    cnt = edge_count.astype(jnp.float32)
    pad = E_PAD - E
    src_p = jnp.concatenate([src, jnp.zeros((pad,), jnp.int32)])
    dst_p = jnp.concatenate([dst, jnp.zeros((pad,), jnp.int32)])
    cnt_p = jnp.concatenate([cnt, jnp.zeros((pad,), jnp.float32)])
    zeros = jnp.zeros((RPT, D), jnp.float32)

    def layer(h, W, b, w0, w1):
        agg = _sc_aggregate(h.reshape(2 * N, D), src_p, dst_p, cnt_p, zeros,
                            with_w=w0 is None)
        if w0 is None:
            w0 = agg[2 * N_PAD:2 * N_PAD + N]
            w1 = agg[3 * N_PAD:3 * N_PAD + N]
        out = _tc_layer(agg[:N], agg[N_PAD:N_PAD + N], w0, w1, h,
                        W[:D], W[D:2 * D], W[2 * D:], b)
        return out, w0, w1

    h1, w0, w1 = layer(x, W1, b1, None, None)
    h2, _, _ = layer(h1, W2, b2, w0, w1)
    return h2


# speedup vs baseline: 2.7164x; 1.3677x over previous
import functools

import jax
import jax.numpy as jnp
from jax import lax
from jax.experimental import pallas as pl
from jax.experimental.pallas import tpu as pltpu
from jax.experimental.pallas import tpu_sc as plsc

N = 10000
E = 160000
D = 128
NC = 2
NT = 16
E_PAD = 163840
EPT = E_PAD // NT
CH = 128
NCH = EPT // CH
WPT = E_PAD // (NC * NT)
WCH = WPT // CH
N_PAD = 10240
RPT = N_PAD // NT
HQ = 40


def _sc_aggregate(x2, src, dst, cnt, zeros, with_w):
    mesh = plsc.VectorSubcoreMesh(core_axis_name="c", subcore_axis_name="s")
    out_rows = (4 if with_w else 2) * N_PAD

    @functools.partial(
        pl.kernel,
        out_type=jax.ShapeDtypeStruct((out_rows, D), jnp.float32),
        mesh=mesh,
        scratch_types=[
            pltpu.VMEM((HQ, CH), jnp.int32),
            pltpu.VMEM((HQ, CH), jnp.int32),
            pltpu.VMEM((HQ, CH), jnp.float32),
            pltpu.VMEM((CH,), jnp.int32),
            pltpu.VMEM((CH,), jnp.int32),
            pltpu.VMEM((CH,), jnp.int32),
            pltpu.VMEM((CH,), jnp.int32),
            pltpu.VMEM((CH, D), jnp.float32),
            pltpu.VMEM((CH, D), jnp.float32),
            pltpu.VMEM_SHARED((N_PAD, D), jnp.float32),
            pltpu.SemaphoreType.DMA,
            pltpu.SemaphoreType.DMA,
            pltpu.SemaphoreType.DMA,
            pltpu.SemaphoreType.DMA,
        ],
    )
    def agg(x2_hbm, src_hbm, dst_hbm, cnt_hbm, z_hbm, out_hbm,
            src_s, dst_s, cnt_s, idx_a, idx_b, dst_a, dst_b, buf_a, buf_b,
            acc, sem_a, sem_b, sc_a, sc_b):
        c = lax.axis_index("c")
        s = lax.axis_index("s")
        pltpu.sync_copy(z_hbm, acc.at[pl.ds(s * RPT, RPT)])
        plsc.subcore_barrier()

        cvec = jnp.full((16,), c, dtype=jnp.int32)

        def build_idx(row, idx_ref):
            for g in range(CH // 16):
                sl = pl.ds(g * 16, 16)
                idx_ref[sl] = src_s[row, sl] * 2 + cvec

        def copy_dst(row, dref):
            for g in range(CH // 16):
                sl = pl.ds(g * 16, 16)
                dref[sl] = dst_s[row, sl]

        def scale(row, buf):
            def group(g, carry):
                c16 = cnt_s[row, pl.ds(g * 16, 16)]
                base = g * 16
                for j in range(16):
                    cvv = jnp.full((16,), c16[j], dtype=jnp.float32)
                    for f in range(D // 16):
                        fsl = pl.ds(f * 16, 16)
                        buf[base + j, fsl] = buf[base + j, fsl] * cvv
                return carry
            lax.fori_loop(0, CH // 16, group, 0)

        def half(hh, carry0):
            pltpu.sync_copy(src_hbm.at[pl.ds(s * NCH + hh * HQ, HQ)], src_s)
            pltpu.sync_copy(dst_hbm.at[pl.ds(s * NCH + hh * HQ, HQ)], dst_s)
            pltpu.sync_copy(cnt_hbm.at[pl.ds(s * NCH + hh * HQ, HQ)], cnt_s)
            build_idx(0, idx_a)
            pltpu.async_copy(x2_hbm.at[idx_a], buf_a, sem_a)
            build_idx(1, idx_b)
            pltpu.async_copy(x2_hbm.at[idx_b], buf_b, sem_b)

            def pair(p, carry):
                i = 2 * p
                pltpu.make_async_copy(x2_hbm.at[idx_a], buf_a, sem_a).wait()
                scale(i, buf_a)
                copy_dst(i, dst_a)
                pltpu.sync_copy(buf_a, acc.at[dst_a], add=True)
                pltpu.make_async_copy(x2_hbm.at[idx_b], buf_b, sem_b).wait()
                scale(i + 1, buf_b)
                copy_dst(i + 1, dst_b)
                pltpu.sync_copy(buf_b, acc.at[dst_b], add=True)
                nxt_a = jnp.minimum(i + 2, HQ - 1)
                nxt_b = jnp.minimum(i + 3, HQ - 1)
                build_idx(nxt_a, idx_a)
                pltpu.async_copy(x2_hbm.at[idx_a], buf_a, sem_a)
                build_idx(nxt_b, idx_b)
                pltpu.async_copy(x2_hbm.at[idx_b], buf_b, sem_b)
                return carry

            lax.fori_loop(0, HQ // 2, pair, 0)
            pltpu.make_async_copy(x2_hbm.at[idx_a], buf_a, sem_a).wait()
            pltpu.make_async_copy(x2_hbm.at[idx_b], buf_b, sem_b).wait()
            return carry0

        lax.fori_loop(0, NCH // HQ, half, 0)
        plsc.subcore_barrier()
        pltpu.sync_copy(acc.at[pl.ds(s * RPT, RPT)],
                        out_hbm.at[pl.ds(c * N_PAD + s * RPT, RPT)])

        if with_w:
            plsc.subcore_barrier()
            pltpu.sync_copy(z_hbm, acc.at[pl.ds(s * RPT, RPT)])
            pltpu.sync_copy(z_hbm.at[pl.ds(0, CH)], buf_a)
            wid = s * NC + c
            pltpu.sync_copy(dst_hbm.at[pl.ds(wid * WCH, WCH)],
                            dst_s.at[pl.ds(0, WCH)])
            pltpu.sync_copy(cnt_hbm.at[pl.ds(wid * WCH, WCH)],
                            cnt_s.at[pl.ds(0, WCH)])
            plsc.subcore_barrier()
            lane0 = jnp.where(lax.iota(jnp.int32, 16) == 0,
                              jnp.full((16,), 1.0, dtype=jnp.float32),
                              jnp.full((16,), 0.0, dtype=jnp.float32))

            def wchunk(i, carry):
                def group(g, carry2):
                    c16 = cnt_s[i, pl.ds(g * 16, 16)]
                    base = g * 16
                    for j in range(16):
                        buf_a[base + j, pl.ds(0, 16)] = lane0 * jnp.full(
                            (16,), c16[j], dtype=jnp.float32)
                    return carry2
                lax.fori_loop(0, CH // 16, group, 0)
                copy_dst(i, dst_a)
                pltpu.sync_copy(buf_a, acc.at[dst_a], add=True)
                return carry

            lax.fori_loop(0, WCH, wchunk, 0)
            plsc.subcore_barrier()
            pltpu.sync_copy(
                acc.at[pl.ds(s * RPT, RPT)],
                out_hbm.at[pl.ds((2 + c) * N_PAD + s * RPT, RPT)])

    return agg(x2, src, dst, cnt, zeros)


def _tc_layer(a0, a1, w0, w1, h, wn0, wn1, wh, b):
    br = 1000

    def body(a0_r, a1_r, w0_r, w1_r, h_r, wn0_r, wn1_r, wh_r, b_r, o_r):
        w = w0_r[:, :1] + w1_r[:, :1]
        inv = 1.0 / jnp.maximum(w, 1.0)
        n0 = a0_r[...] * inv
        n1 = a1_r[...] * inv
        z = (jnp.dot(n0, wn0_r[...], preferred_element_type=jnp.float32)
             + jnp.dot(n1, wn1_r[...], preferred_element_type=jnp.float32)
             + jnp.dot(h_r[...], wh_r[...], preferred_element_type=jnp.float32)
             + b_r[...])
        z = jnp.maximum(z, 0.0)
        ssum = jnp.sum(z * z, axis=1, keepdims=True)
        o_r[...] = z * lax.rsqrt(jnp.where(ssum == 0.0, 1.0, ssum))

    return pl.pallas_call(
        body,
        grid=(N // br,),
        in_specs=[
            pl.BlockSpec((br, D), lambda i: (i, 0)),
            pl.BlockSpec((br, D), lambda i: (i, 0)),
            pl.BlockSpec((br, D), lambda i: (i, 0)),
            pl.BlockSpec((br, D), lambda i: (i, 0)),
            pl.BlockSpec((br, 2 * D), lambda i: (i, 0)),
            pl.BlockSpec((D, 2 * D), lambda i: (0, 0)),
            pl.BlockSpec((D, 2 * D), lambda i: (0, 0)),
            pl.BlockSpec((2 * D, 2 * D), lambda i: (0, 0)),
            pl.BlockSpec((1, 2 * D), lambda i: (0, 0)),
        ],
        out_specs=pl.BlockSpec((br, 2 * D), lambda i: (i, 0)),
        out_shape=jax.ShapeDtypeStruct((N, 2 * D), jnp.float32),
    )(a0, a1, w0, w1, h, wn0, wn1, wh, b.reshape(1, 2 * D))


def kernel(x, edge_index, edge_count, W1, b1, W2, b2):
    src = edge_index[0].astype(jnp.int32)
    dst = edge_index[1].astype(jnp.int32)
    cnt = edge_count.astype(jnp.float32)
    pad = E_PAD - E
    src_p = jnp.concatenate([src, jnp.zeros((pad,), jnp.int32)])
    dst_p = jnp.concatenate([dst, jnp.zeros((pad,), jnp.int32)])
    cnt_p = jnp.concatenate([cnt, jnp.zeros((pad,), jnp.float32)])
    src_p = src_p.reshape(E_PAD // CH, CH)
    dst_p = dst_p.reshape(E_PAD // CH, CH)
    cnt_p = cnt_p.reshape(E_PAD // CH, CH)
    zeros = jnp.zeros((RPT, D), jnp.float32)

    def layer(h, W, b, w0, w1):
        agg = _sc_aggregate(h.reshape(2 * N, D), src_p, dst_p, cnt_p, zeros,
                            with_w=w0 is None)
        if w0 is None:
            w0 = agg[2 * N_PAD:2 * N_PAD + N]
            w1 = agg[3 * N_PAD:3 * N_PAD + N]
        out = _tc_layer(agg[:N], agg[N_PAD:N_PAD + N], w0, w1, h,
                        W[:D], W[D:2 * D], W[2 * D:], b)
        return out, w0, w1

    h1, w0, w1 = layer(x, W1, b1, None, None)
    h2, _, _ = layer(h1, W2, b2, w0, w1)
    return h2


# speedup vs baseline: 3.1816x; 1.1712x over previous
import functools

import jax
import jax.numpy as jnp
from jax import lax
from jax.experimental import pallas as pl
from jax.experimental.pallas import tpu as pltpu
from jax.experimental.pallas import tpu_sc as plsc

N = 10000
E = 160000
D = 128
NC = 2
NT = 16
E_PAD = 163840
EPT = E_PAD // NT
CH = 128
NCH = EPT // CH
WPT = E_PAD // (NC * NT)
WCH = WPT // CH
N_PAD = 10240
RPT = N_PAD // NT
HQ = 40


def _sc_aggregate(x2, src, dst, cnt, zeros, with_w):
    mesh = plsc.VectorSubcoreMesh(core_axis_name="c", subcore_axis_name="s")
    out_rows = (4 if with_w else 2) * N_PAD

    @functools.partial(
        pl.kernel,
        out_type=jax.ShapeDtypeStruct((out_rows, D), jnp.float32),
        mesh=mesh,
        scratch_types=[
            pltpu.VMEM((HQ, CH), jnp.int32),
            pltpu.VMEM((HQ, CH), jnp.int32),
            pltpu.VMEM((HQ, CH), jnp.float32),
            pltpu.VMEM((CH,), jnp.int32),
            pltpu.VMEM((CH,), jnp.int32),
            pltpu.VMEM((CH,), jnp.int32),
            pltpu.VMEM((CH,), jnp.int32),
            pltpu.VMEM((CH, D), jnp.float32),
            pltpu.VMEM((CH, D), jnp.float32),
            pltpu.VMEM_SHARED((N_PAD, D), jnp.float32),
            pltpu.SemaphoreType.DMA,
            pltpu.SemaphoreType.DMA,
            pltpu.SemaphoreType.DMA,
            pltpu.SemaphoreType.DMA,
        ],
    )
    def agg(x2_hbm, src_hbm, dst_hbm, cnt_hbm, z_hbm, out_hbm,
            src_s, dst_s, cnt_s, idx_a, idx_b, dst_a, dst_b, buf_a, buf_b,
            acc, sem_a, sem_b, sc_a, sc_b):
        c = lax.axis_index("c")
        s = lax.axis_index("s")
        pltpu.sync_copy(z_hbm, acc.at[pl.ds(s * RPT, RPT)])
        plsc.subcore_barrier()

        cvec = jnp.full((16,), c, dtype=jnp.int32)

        def build_idx(row, idx_ref):
            for g in range(CH // 16):
                sl = pl.ds(g * 16, 16)
                idx_ref[sl] = src_s[row, sl] * 2 + cvec

        def copy_dst(row, dref):
            for g in range(CH // 16):
                sl = pl.ds(g * 16, 16)
                dref[sl] = dst_s[row, sl]

        def scale(row, buf):
            def group(g, carry):
                c16 = cnt_s[row, pl.ds(g * 16, 16)]
                base = g * 16
                for j in range(16):
                    cvv = jnp.full((16,), c16[j], dtype=jnp.float32)
                    for f in range(D // 16):
                        fsl = pl.ds(f * 16, 16)
                        buf[base + j, fsl] = buf[base + j, fsl] * cvv
                return carry
            lax.fori_loop(0, CH // 16, group, 0)

        def half(hh, carry0):
            pltpu.sync_copy(src_hbm.at[pl.ds(s * NCH + hh * HQ, HQ)], src_s)
            pltpu.sync_copy(dst_hbm.at[pl.ds(s * NCH + hh * HQ, HQ)], dst_s)
            pltpu.sync_copy(cnt_hbm.at[pl.ds(s * NCH + hh * HQ, HQ)], cnt_s)
            build_idx(0, idx_a)
            pltpu.async_copy(x2_hbm.at[idx_a], buf_a, sem_a)
            build_idx(1, idx_b)
            pltpu.async_copy(x2_hbm.at[idx_b], buf_b, sem_b)

            def pair(p, carry):
                i = 2 * p
                pltpu.make_async_copy(x2_hbm.at[idx_a], buf_a, sem_a).wait()
                scale(i, buf_a)
                copy_dst(i, dst_a)
                pltpu.async_copy(buf_a, acc.at[dst_a], sc_a, add=True)
                pltpu.make_async_copy(x2_hbm.at[idx_b], buf_b, sem_b).wait()
                scale(i + 1, buf_b)
                copy_dst(i + 1, dst_b)
                pltpu.async_copy(buf_b, acc.at[dst_b], sc_b, add=True)
                nxt_a = jnp.minimum(i + 2, HQ - 1)
                nxt_b = jnp.minimum(i + 3, HQ - 1)
                build_idx(nxt_a, idx_a)
                pltpu.make_async_copy(buf_a, acc.at[dst_a], sc_a).wait()
                pltpu.async_copy(x2_hbm.at[idx_a], buf_a, sem_a)
                build_idx(nxt_b, idx_b)
                pltpu.make_async_copy(buf_b, acc.at[dst_b], sc_b).wait()
                pltpu.async_copy(x2_hbm.at[idx_b], buf_b, sem_b)
                return carry

            lax.fori_loop(0, HQ // 2, pair, 0)
            pltpu.make_async_copy(x2_hbm.at[idx_a], buf_a, sem_a).wait()
            pltpu.make_async_copy(x2_hbm.at[idx_b], buf_b, sem_b).wait()
            return carry0

        lax.fori_loop(0, NCH // HQ, half, 0)
        plsc.subcore_barrier()
        pltpu.sync_copy(acc.at[pl.ds(s * RPT, RPT)],
                        out_hbm.at[pl.ds(c * N_PAD + s * RPT, RPT)])

        if with_w:
            plsc.subcore_barrier()
            pltpu.sync_copy(z_hbm, acc.at[pl.ds(s * RPT, RPT)])
            pltpu.sync_copy(z_hbm.at[pl.ds(0, CH)], buf_a)
            wid = s * NC + c
            pltpu.sync_copy(dst_hbm.at[pl.ds(wid * WCH, WCH)],
                            dst_s.at[pl.ds(0, WCH)])
            pltpu.sync_copy(cnt_hbm.at[pl.ds(wid * WCH, WCH)],
                            cnt_s.at[pl.ds(0, WCH)])
            plsc.subcore_barrier()
            lane0 = jnp.where(lax.iota(jnp.int32, 16) == 0,
                              jnp.full((16,), 1.0, dtype=jnp.float32),
                              jnp.full((16,), 0.0, dtype=jnp.float32))

            def wchunk(i, carry):
                def group(g, carry2):
                    c16 = cnt_s[i, pl.ds(g * 16, 16)]
                    base = g * 16
                    for j in range(16):
                        buf_a[base + j, pl.ds(0, 16)] = lane0 * jnp.full(
                            (16,), c16[j], dtype=jnp.float32)
                    return carry2
                lax.fori_loop(0, CH // 16, group, 0)
                copy_dst(i, dst_a)
                pltpu.sync_copy(buf_a, acc.at[dst_a], add=True)
                return carry

            lax.fori_loop(0, WCH, wchunk, 0)
            plsc.subcore_barrier()
            pltpu.sync_copy(
                acc.at[pl.ds(s * RPT, RPT)],
                out_hbm.at[pl.ds((2 + c) * N_PAD + s * RPT, RPT)])

    return agg(x2, src, dst, cnt, zeros)


def _tc_layer(a0, a1, w0, w1, h, wn0, wn1, wh, b):
    br = 1000

    def body(a0_r, a1_r, w0_r, w1_r, h_r, wn0_r, wn1_r, wh_r, b_r, o_r):
        w = w0_r[:, :1] + w1_r[:, :1]
        inv = 1.0 / jnp.maximum(w, 1.0)
        n0 = a0_r[...] * inv
        n1 = a1_r[...] * inv
        z = (jnp.dot(n0, wn0_r[...], preferred_element_type=jnp.float32)
             + jnp.dot(n1, wn1_r[...], preferred_element_type=jnp.float32)
             + jnp.dot(h_r[...], wh_r[...], preferred_element_type=jnp.float32)
             + b_r[...])
        z = jnp.maximum(z, 0.0)
        ssum = jnp.sum(z * z, axis=1, keepdims=True)
        o_r[...] = z * lax.rsqrt(jnp.where(ssum == 0.0, 1.0, ssum))

    return pl.pallas_call(
        body,
        grid=(N // br,),
        in_specs=[
            pl.BlockSpec((br, D), lambda i: (i, 0)),
            pl.BlockSpec((br, D), lambda i: (i, 0)),
            pl.BlockSpec((br, D), lambda i: (i, 0)),
            pl.BlockSpec((br, D), lambda i: (i, 0)),
            pl.BlockSpec((br, 2 * D), lambda i: (i, 0)),
            pl.BlockSpec((D, 2 * D), lambda i: (0, 0)),
            pl.BlockSpec((D, 2 * D), lambda i: (0, 0)),
            pl.BlockSpec((2 * D, 2 * D), lambda i: (0, 0)),
            pl.BlockSpec((1, 2 * D), lambda i: (0, 0)),
        ],
        out_specs=pl.BlockSpec((br, 2 * D), lambda i: (i, 0)),
        out_shape=jax.ShapeDtypeStruct((N, 2 * D), jnp.float32),
    )(a0, a1, w0, w1, h, wn0, wn1, wh, b.reshape(1, 2 * D))


def kernel(x, edge_index, edge_count, W1, b1, W2, b2):
    src = edge_index[0].astype(jnp.int32)
    dst = edge_index[1].astype(jnp.int32)
    cnt = edge_count.astype(jnp.float32)
    pad = E_PAD - E
    src_p = jnp.concatenate([src, jnp.zeros((pad,), jnp.int32)])
    dst_p = jnp.concatenate([dst, jnp.zeros((pad,), jnp.int32)])
    cnt_p = jnp.concatenate([cnt, jnp.zeros((pad,), jnp.float32)])
    src_p = src_p.reshape(E_PAD // CH, CH)
    dst_p = dst_p.reshape(E_PAD // CH, CH)
    cnt_p = cnt_p.reshape(E_PAD // CH, CH)
    zeros = jnp.zeros((RPT, D), jnp.float32)

    def layer(h, W, b, w0, w1):
        agg = _sc_aggregate(h.reshape(2 * N, D), src_p, dst_p, cnt_p, zeros,
                            with_w=w0 is None)
        if w0 is None:
            w0 = agg[2 * N_PAD:2 * N_PAD + N]
            w1 = agg[3 * N_PAD:3 * N_PAD + N]
        out = _tc_layer(agg[:N], agg[N_PAD:N_PAD + N], w0, w1, h,
                        W[:D], W[D:2 * D], W[2 * D:], b)
        return out, w0, w1

    h1, w0, w1 = layer(x, W1, b1, None, None)
    h2, _, _ = layer(h1, W2, b2, w0, w1)
    return h2
